# baseline (speedup 1.0000x reference)
import functools

import jax
import jax.numpy as jnp
from jax.experimental import pallas as pl
from jax.experimental.pallas import tpu as pltpu

_SEG = 10000
_CC = 2000
_NSLOT = 2


def _gat_chain_body(loc_hbm, ws_ref, wr_ref, bias_ref, o_hbm,
                    in_buf, out_buf, row_buf, in_sem, out_sem):
    d = ws_ref.shape[0]
    h = ws_ref.shape[1] // d
    nb, l, _ = loc_hbm.shape
    seg = in_buf.shape[1]
    nseg_per_b = l // seg
    nseg = nb * nseg_per_b
    ncc = seg // _CC
    inv_h = 1.0 / h

    a = ws_ref[:, 0:d]
    r = wr_ref[:, 0:d]
    for i in range(1, h):
        a = a + ws_ref[:, i * d:(i + 1) * d]
        r = r + wr_ref[:, i * d:(i + 1) * d]
    a = a * inv_h
    r = r * inv_h
    c = jnp.mean(bias_ref[...], axis=0, keepdims=True)

    def in_copy(g):
        bi, si = divmod(g, nseg_per_b)
        return pltpu.make_async_copy(
            loc_hbm.at[bi, pl.ds(si * seg, seg), :], in_buf.at[g % _NSLOT],
            in_sem.at[g % _NSLOT])

    def out_copy(g):
        bi, si = divmod(g, nseg_per_b)
        return pltpu.make_async_copy(
            out_buf.at[g % _NSLOT], o_hbm.at[bi, pl.ds(si * seg, seg), :],
            out_sem.at[g % _NSLOT])

    for g0 in range(min(_NSLOT, nseg)):
        in_copy(g0).start()
    for g in range(nseg):
        s = g % _NSLOT
        _, si = divmod(g, nseg_per_b)
        in_copy(g).wait()
        if g >= _NSLOT:
            out_copy(g - _NSLOT).wait()
        for j in range(ncc):
            r0 = j * _CC
            x = in_buf[s, r0:r0 + _CC, :]
            y = jnp.dot(x, a, preferred_element_type=jnp.float32)
            z = jnp.dot(x, r, preferred_element_type=jnp.float32)
            y_shift = pltpu.roll(y, 1, axis=0)
            if j == 0 and si == 0:
                first = x[0:1, :]
            else:
                if j == 0:
                    prev = row_buf[(g - 1) % _NSLOT, 7:8, :]
                else:
                    prev = in_buf[s, r0 - 1:r0, :]
                first = (jnp.dot(prev, a, preferred_element_type=jnp.float32)
                         + z[0:1, :] + c)
            row = jax.lax.broadcasted_iota(jnp.int32, y.shape, 0)
            res = jnp.where(row == 0, first, y_shift + z + c)
            out_buf[s, r0:r0 + _CC, :] = res
        row_buf[s] = in_buf[s, seg - 8:seg, :]
        out_copy(g).start()
        if g + _NSLOT < nseg:
            in_copy(g + _NSLOT).start()
    for g0 in range(max(nseg - _NSLOT, 0), nseg):
        out_copy(g0).wait()


@functools.partial(jax.jit, static_argnames=())
def kernel(batch, loc, W_src, W_dst, attn_l, attn_r, W_res, bias):
    del batch, W_dst, attn_l, attn_r
    b, l, d = loc.shape
    hd = W_src.shape[1]
    h = hd // d
    seg = _SEG if (l % _SEG == 0) else l

    bias2d = bias.reshape(h, d)

    out = pl.pallas_call(
        _gat_chain_body,
        in_specs=[
            pl.BlockSpec(memory_space=pl.ANY),
            pl.BlockSpec((d, hd), lambda: (0, 0)),
            pl.BlockSpec((d, hd), lambda: (0, 0)),
            pl.BlockSpec((h, d), lambda: (0, 0)),
        ],
        out_specs=pl.BlockSpec(memory_space=pl.ANY),
        out_shape=jax.ShapeDtypeStruct((b, l, d), jnp.float32),
        scratch_shapes=[
            pltpu.VMEM((_NSLOT, seg, d), jnp.float32),
            pltpu.VMEM((_NSLOT, seg, d), jnp.float32),
            pltpu.VMEM((_NSLOT, 8, d), jnp.float32),
            pltpu.SemaphoreType.DMA((_NSLOT,)),
            pltpu.SemaphoreType.DMA((_NSLOT,)),
        ],
    )(loc, W_src, W_res, bias2d)
    return out


# speedup vs baseline: 1.1111x; 1.1111x over previous
import functools

import jax
import jax.numpy as jnp
from jax.experimental import pallas as pl
from jax.experimental.pallas import tpu as pltpu

_CC = 2000
_IN_SLOTS = 3
_OUT_SLOTS = 8


def _gat_chain_body(loc_hbm, ws_ref, wr_ref, bias_ref, o_hbm,
                    in_buf, out_buf, in_sem, out_sem):
    d = ws_ref.shape[0]
    h = ws_ref.shape[1] // d
    nb, l, _ = loc_hbm.shape
    ncc = l // _CC
    n_out = nb * ncc
    inv_h = 1.0 / h

    a = ws_ref[:, 0:d]
    r = wr_ref[:, 0:d]
    for i in range(1, h):
        a = a + ws_ref[:, i * d:(i + 1) * d]
        r = r + wr_ref[:, i * d:(i + 1) * d]
    a = a * inv_h
    r = r * inv_h
    c = jnp.mean(bias_ref[...], axis=0, keepdims=True)

    def in_copy(g):
        return pltpu.make_async_copy(
            loc_hbm.at[g], in_buf.at[g % _IN_SLOTS], in_sem.at[g % _IN_SLOTS])

    def out_copy(k):
        bi, j = divmod(k, ncc)
        return pltpu.make_async_copy(
            out_buf.at[k % _OUT_SLOTS], o_hbm.at[bi, pl.ds(j * _CC, _CC), :],
            out_sem.at[k % _OUT_SLOTS])

    for g0 in range(min(_IN_SLOTS, nb)):
        in_copy(g0).start()
    for g in range(nb):
        s = g % _IN_SLOTS
        in_copy(g).wait()
        for j in range(ncc):
            k = g * ncc + j
            r0 = j * _CC
            x = in_buf[s, r0:r0 + _CC, :]
            y = jnp.dot(x, a, preferred_element_type=jnp.float32)
            z = jnp.dot(x, r, preferred_element_type=jnp.float32)
            y_shift = pltpu.roll(y, 1, axis=0)
            if j == 0:
                first = x[0:1, :]
            else:
                prev = in_buf[s, r0 - 1:r0, :]
                first = (jnp.dot(prev, a, preferred_element_type=jnp.float32)
                         + z[0:1, :] + c)
            row = jax.lax.broadcasted_iota(jnp.int32, y.shape, 0)
            res = jnp.where(row == 0, first, y_shift + z + c)
            if k >= _OUT_SLOTS:
                out_copy(k - _OUT_SLOTS).wait()
            out_buf[k % _OUT_SLOTS] = res
            out_copy(k).start()
        if g + _IN_SLOTS < nb:
            in_copy(g + _IN_SLOTS).start()
    for k0 in range(max(n_out - _OUT_SLOTS, 0), n_out):
        out_copy(k0).wait()


@functools.partial(jax.jit, static_argnames=())
def kernel(batch, loc, W_src, W_dst, attn_l, attn_r, W_res, bias):
    del batch, W_dst, attn_l, attn_r
    b, l, d = loc.shape
    hd = W_src.shape[1]
    h = hd // d
    cc = _CC if (l % _CC == 0) else l

    bias2d = bias.reshape(h, d)

    out = pl.pallas_call(
        _gat_chain_body,
        in_specs=[
            pl.BlockSpec(memory_space=pl.ANY),
            pl.BlockSpec((d, hd), lambda: (0, 0)),
            pl.BlockSpec((d, hd), lambda: (0, 0)),
            pl.BlockSpec((h, d), lambda: (0, 0)),
        ],
        out_specs=pl.BlockSpec(memory_space=pl.ANY),
        out_shape=jax.ShapeDtypeStruct((b, l, d), jnp.float32),
        scratch_shapes=[
            pltpu.VMEM((_IN_SLOTS, l, d), jnp.float32),
            pltpu.VMEM((_OUT_SLOTS, cc, d), jnp.float32),
            pltpu.SemaphoreType.DMA((_IN_SLOTS,)),
            pltpu.SemaphoreType.DMA((_OUT_SLOTS,)),
        ],
    )(loc, W_src, W_res, bias2d)
    return out


# speedup vs baseline: 1.1466x; 1.0320x over previous
import functools

import jax
import jax.numpy as jnp
from jax.experimental import pallas as pl
from jax.experimental.pallas import tpu as pltpu

_CC = 2000
_IN_SLOTS = 2
_OUT_SLOTS = 8


def _gat_chain_body(loc_hbm, ws_ref, wr_ref, bias_ref, o_hbm,
                    in_buf, out_buf, in_sem, out_sem):
    d = ws_ref.shape[0]
    h = ws_ref.shape[1] // d
    nb, l, _ = loc_hbm.shape
    ncc = l // _CC
    n_out = nb * ncc
    inv_h = 1.0 / h

    a = ws_ref[:, 0:d]
    r = wr_ref[:, 0:d]
    for i in range(1, h):
        a = a + ws_ref[:, i * d:(i + 1) * d]
        r = r + wr_ref[:, i * d:(i + 1) * d]
    a = a * inv_h
    r = r * inv_h
    c = jnp.mean(bias_ref[...], axis=0, keepdims=True)

    def in_copy(g, j):
        p = g % _IN_SLOTS
        return pltpu.make_async_copy(
            loc_hbm.at[g, pl.ds(j * _CC, _CC), :],
            in_buf.at[p, pl.ds(j * _CC, _CC), :],
            in_sem.at[p, j])

    def out_copy(k):
        bi, j = divmod(k, ncc)
        return pltpu.make_async_copy(
            out_buf.at[k % _OUT_SLOTS], o_hbm.at[bi, pl.ds(j * _CC, _CC), :],
            out_sem.at[k % _OUT_SLOTS])

    for g0 in range(min(_IN_SLOTS, nb)):
        for j0 in range(ncc):
            in_copy(g0, j0).start()
    for g in range(nb):
        p = g % _IN_SLOTS
        for j in range(ncc):
            k = g * ncc + j
            r0 = j * _CC
            in_copy(g, j).wait()
            x = in_buf[p, r0:r0 + _CC, :]
            y = jnp.dot(x, a, preferred_element_type=jnp.float32)
            z = jnp.dot(x, r, preferred_element_type=jnp.float32)
            y_shift = pltpu.roll(y, 1, axis=0)
            if j == 0:
                first = x[0:1, :]
            else:
                prev = in_buf[p, r0 - 1:r0, :]
                first = (jnp.dot(prev, a, preferred_element_type=jnp.float32)
                         + z[0:1, :] + c)
            row = jax.lax.broadcasted_iota(jnp.int32, y.shape, 0)
            res = jnp.where(row == 0, first, y_shift + z + c)
            if k >= _OUT_SLOTS:
                out_copy(k - _OUT_SLOTS).wait()
            out_buf[k % _OUT_SLOTS] = res
            out_copy(k).start()
        if g + _IN_SLOTS < nb:
            for j in range(ncc):
                in_copy(g + _IN_SLOTS, j).start()
    for k0 in range(max(n_out - _OUT_SLOTS, 0), n_out):
        out_copy(k0).wait()


@functools.partial(jax.jit, static_argnames=())
def kernel(batch, loc, W_src, W_dst, attn_l, attn_r, W_res, bias):
    del batch, W_dst, attn_l, attn_r
    b, l, d = loc.shape
    hd = W_src.shape[1]
    h = hd // d
    cc = _CC if (l % _CC == 0) else l
    ncc = l // cc

    bias2d = bias.reshape(h, d)

    out = pl.pallas_call(
        _gat_chain_body,
        in_specs=[
            pl.BlockSpec(memory_space=pl.ANY),
            pl.BlockSpec((d, hd), lambda: (0, 0)),
            pl.BlockSpec((d, hd), lambda: (0, 0)),
            pl.BlockSpec((h, d), lambda: (0, 0)),
        ],
        out_specs=pl.BlockSpec(memory_space=pl.ANY),
        out_shape=jax.ShapeDtypeStruct((b, l, d), jnp.float32),
        scratch_shapes=[
            pltpu.VMEM((_IN_SLOTS, l, d), jnp.float32),
            pltpu.VMEM((_OUT_SLOTS, cc, d), jnp.float32),
            pltpu.SemaphoreType.DMA((_IN_SLOTS, ncc)),
            pltpu.SemaphoreType.DMA((_OUT_SLOTS,)),
        ],
    )(loc, W_src, W_res, bias2d)
    return out


# speedup vs baseline: 1.1694x; 1.0199x over previous
import functools

import jax
import jax.numpy as jnp
from jax.experimental import pallas as pl
from jax.experimental.pallas import tpu as pltpu

_CHUNK = 2000
_NBUF = 16


def _gat_chain_body(loc_hbm, ws_ref, wr_ref, bias_ref, o_hbm,
                    in_buf, out_buf, row_buf, in_sem, out_sem):
    d = ws_ref.shape[0]
    h = ws_ref.shape[1] // d
    nb, l, _ = loc_hbm.shape
    cc = in_buf.shape[1]
    nchunk = l // cc
    n = nb * nchunk
    nbuf = in_buf.shape[0]
    inv_h = 1.0 / h

    a = ws_ref[:, 0:d]
    r = wr_ref[:, 0:d]
    for i in range(1, h):
        a = a + ws_ref[:, i * d:(i + 1) * d]
        r = r + wr_ref[:, i * d:(i + 1) * d]
    a = a * inv_h
    r = r * inv_h
    c = jnp.mean(bias_ref[...], axis=0, keepdims=True)

    def in_copy(k):
        bi, j = divmod(k, nchunk)
        return pltpu.make_async_copy(
            loc_hbm.at[bi, pl.ds(j * cc, cc), :], in_buf.at[k % _NBUF],
            in_sem.at[k % _NBUF])

    def out_copy(k):
        bi, j = divmod(k, nchunk)
        return pltpu.make_async_copy(
            out_buf.at[k % _NBUF], o_hbm.at[bi, pl.ds(j * cc, cc), :],
            out_sem.at[k % _NBUF])

    for k0 in range(min(nbuf, n)):
        in_copy(k0).start()
    for k in range(n):
        s = k % nbuf
        _, j = divmod(k, nchunk)
        in_copy(k).wait()
        x = in_buf[s]
        y = jnp.dot(x, a, preferred_element_type=jnp.float32)
        z = jnp.dot(x, r, preferred_element_type=jnp.float32)
        y_shift = pltpu.roll(y, 1, axis=0)
        if j == 0:
            first = x[0:1, :]
        else:
            prev = row_buf[(k - 1) % nbuf, 7:8, :]
            first = (jnp.dot(prev, a, preferred_element_type=jnp.float32)
                     + z[0:1, :] + c)
        row = jax.lax.broadcasted_iota(jnp.int32, y.shape, 0)
        res = jnp.where(row == 0, first, y_shift + z + c)
        if k >= nbuf:
            out_copy(k - nbuf).wait()
        out_buf[s] = res
        row_buf[s] = x[cc - 8:cc, :]
        out_copy(k).start()
        if k + nbuf < n:
            in_copy(k + nbuf).start()
    for k0 in range(max(n - nbuf, 0), n):
        out_copy(k0).wait()


@functools.partial(jax.jit, static_argnames=())
def kernel(batch, loc, W_src, W_dst, attn_l, attn_r, W_res, bias):
    del batch, W_dst, attn_l, attn_r
    b, l, d = loc.shape
    hd = W_src.shape[1]
    h = hd // d
    cc = _CHUNK if (l % _CHUNK == 0 and _CHUNK % 8 == 0) else l

    bias2d = bias.reshape(h, d)

    out = pl.pallas_call(
        _gat_chain_body,
        in_specs=[
            pl.BlockSpec(memory_space=pl.ANY),
            pl.BlockSpec((d, hd), lambda: (0, 0)),
            pl.BlockSpec((d, hd), lambda: (0, 0)),
            pl.BlockSpec((h, d), lambda: (0, 0)),
        ],
        out_specs=pl.BlockSpec(memory_space=pl.ANY),
        out_shape=jax.ShapeDtypeStruct((b, l, d), jnp.float32),
        scratch_shapes=[
            pltpu.VMEM((_NBUF, cc, d), jnp.float32),
            pltpu.VMEM((_NBUF, cc, d), jnp.float32),
            pltpu.VMEM((_NBUF, 8, d), jnp.float32),
            pltpu.SemaphoreType.DMA((_NBUF,)),
            pltpu.SemaphoreType.DMA((_NBUF,)),
        ],
    )(loc, W_src, W_res, bias2d)
    return out
